# baseline (speedup 1.0000x reference)
import functools

import jax
import jax.numpy as jnp
from jax import lax
from jax.experimental import pallas as pl
from jax.experimental.pallas import tpu as pltpu
from jax.experimental.pallas import tpu_sc as plsc

D = 32
BLK = 128
NC, NS = 2, 16
NW = NC * NS
NSLAB = D // 8


@functools.lru_cache(maxsize=None)
def _make_gather(F: int, Bt: int, V: int):
  n_blocks = F * (Bt // BLK)
  assert n_blocks % NW == 0
  j_per_w = n_blocks // NW
  nbh = Bt // BLK
  mesh = plsc.VectorSubcoreMesh(core_axis_name="c", subcore_axis_name="s")

  @functools.partial(
      pl.kernel,
      out_type=jax.ShapeDtypeStruct((F, NSLAB, nbh, 8, BLK), jnp.float32),
      mesh=mesh,
      compiler_params=pltpu.CompilerParams(
          use_tc_tiling_on_sc=False, needs_layout_passes=False),
      scratch_types=[
          pltpu.VMEM((2, BLK), jnp.int32),
          pltpu.VMEM((2, BLK, D), jnp.float32),
          pltpu.VMEM((2, NSLAB, 8, BLK), jnp.float32),
          pltpu.SemaphoreType.DMA,
          pltpu.SemaphoreType.DMA,
          pltpu.SemaphoreType.DMA,
      ],
  )
  def k(idx_hbm, table_hbm, out_hbm, idx_v, rows_v, t_v, isem, gsem, psem):
    wid = lax.axis_index("s") * NC + lax.axis_index("c")

    def blk_of(j):
      g = wid + NW * j
      f = g // nbh
      bh = g - f * nbh
      return f, bh

    def idx_slice(j):
      f, bh = blk_of(j)
      return idx_hbm.at[pl.ds(f * Bt + bh * BLK, BLK)]

    def start_idx(j, slot):
      return pltpu.async_copy(idx_slice(j), idx_v.at[slot], isem)

    def start_gather(slot):
      return pltpu.async_copy(
          table_hbm.at[idx_v.at[slot]], rows_v.at[slot], gsem)

    def start_puts(j, slot):
      f, bh = blk_of(j)
      return [
          pltpu.async_copy(t_v.at[slot, si], out_hbm.at[f, si, bh], psem)
          for si in range(NSLAB)
      ]

    def wait_puts(j, slot):
      f, bh = blk_of(j)
      for si in range(NSLAB):
        pltpu.make_async_copy(
            t_v.at[slot, si], out_hbm.at[f, si, bh], psem).wait()

    def transpose(slot):
      src = rows_v.at[slot]
      lanes = lax.iota(jnp.int32, 16)

      def c_body(c, _):
        si = c // 8
        cl = c - si * 8
        cvec = jnp.full((16,), c, jnp.int32)
        for chunk in range(BLK // 16):
          bl = lanes + (chunk * 16)
          vec = plsc.load_gather(src, [bl, cvec])
          t_v[slot, si, cl, pl.ds(chunk * 16, 16)] = vec
        return 0

      lax.fori_loop(0, D, c_body, 0)

    pltpu.sync_copy(idx_slice(0), idx_v.at[0])
    start_gather(0)
    start_idx(1, 1)

    def body(j, _):
      s = lax.rem(j, 2)
      p = 1 - s
      pltpu.make_async_copy(idx_slice(j), idx_v.at[s], isem).wait()
      pltpu.make_async_copy(
          table_hbm.at[idx_v.at[p]], rows_v.at[p], gsem).wait()
      start_gather(s)

      @pl.when(j <= j_per_w - 2)
      def _():
        start_idx(j + 1, p)

      @pl.when(j >= 2)
      def _():
        wait_puts(j - 2, p)

      transpose(p)
      start_puts(j - 1, p)
      return 0

    lax.fori_loop(1, j_per_w, body, 0)

    last = j_per_w - 1
    sl = last % 2
    pltpu.make_async_copy(
        table_hbm.at[idx_v.at[sl]], rows_v.at[sl], gsem).wait()
    transpose(sl)
    start_puts(last, sl)
    wait_puts(last - 1, 1 - sl)
    wait_puts(last, sl)

  return k


def kernel(x, table):
  Bt, F = x.shape
  V, d = table.shape
  assert d == D and Bt % BLK == 0
  idx = jnp.swapaxes(x, 0, 1).reshape(F * Bt).astype(jnp.int32)
  out5 = _make_gather(F, Bt, V)(idx, table)
  return out5.transpose(2, 4, 0, 1, 3).reshape(Bt, F, D)
